# baseline (speedup 1.0000x reference)
import numpy as np
import jax
import jax.numpy as jnp
from jax.experimental import pallas as pl
from jax.experimental.pallas import tpu as pltpu

_ALPHA = 4


def _gather_body(frames_ref, slow_ref, fast_ref):
    del fast_ref
    slow_ref[...] = frames_ref[...]


def kernel(frames):
    C, T, H, W = frames.shape
    n_slow = T // _ALPHA
    idx = np.linspace(0.0, T - 1, n_slow).astype(np.int32)
    assert all(int(i) == (s * (T - 1)) // (n_slow - 1) for s, i in enumerate(idx))

    def in_map(s):
        return (0, (s * (T - 1)) // (n_slow - 1), 0, 0)

    def out_map(s):
        return (0, s, 0, 0)

    slow, fast = pl.pallas_call(
        _gather_body,
        grid=(n_slow,),
        in_specs=[pl.BlockSpec((C, 1, H, W), in_map)],
        out_specs=[
            pl.BlockSpec((C, 1, H, W), out_map),
            pl.BlockSpec(memory_space=pl.ANY),
        ],
        out_shape=[
            jax.ShapeDtypeStruct((C, n_slow, H, W), frames.dtype),
            jax.ShapeDtypeStruct((C, T, H, W), frames.dtype),
        ],
        input_output_aliases={0: 1},
    )(frames)
    return (slow, fast)


# speedup vs baseline: 1.2171x; 1.2171x over previous
import numpy as np
import jax
import jax.numpy as jnp
from jax.experimental import pallas as pl
from jax.experimental.pallas import tpu as pltpu

_ALPHA = 4
_GROUP = 8
_SLOW_PER_GROUP = _GROUP // _ALPHA


def _pack_body(frames_ref, slow_ref, fast_ref):
    g = pl.program_id(0)
    n_slow = pl.num_programs(0) * _SLOW_PER_GROUP
    T = n_slow * _ALPHA
    fast_ref[...] = frames_ref[...]
    for u in range(_SLOW_PER_GROUP):
        s = g * _SLOW_PER_GROUP + u
        off = (s * (T - 1)) // (n_slow - 1) - _GROUP * g
        slow_ref[:, pl.ds(u, 1), :, :] = frames_ref[:, pl.ds(off, 1), :, :]


def kernel(frames):
    C, T, H, W = frames.shape
    n_slow = T // _ALPHA
    n_groups = T // _GROUP
    idx = np.linspace(0.0, T - 1, n_slow).astype(np.int32)
    assert all(int(i) == (s * (T - 1)) // (n_slow - 1) for s, i in enumerate(idx))
    assert all(_GROUP * (s // _SLOW_PER_GROUP) <= int(i) < _GROUP * (s // _SLOW_PER_GROUP + 1)
               for s, i in enumerate(idx))

    def group_map(g):
        return (0, g, 0, 0)

    slow, fast = pl.pallas_call(
        _pack_body,
        grid=(n_groups,),
        in_specs=[pl.BlockSpec((C, _GROUP, H, W), group_map)],
        out_specs=[
            pl.BlockSpec((C, _SLOW_PER_GROUP, H, W), group_map),
            pl.BlockSpec((C, _GROUP, H, W), group_map),
        ],
        out_shape=[
            jax.ShapeDtypeStruct((C, n_slow, H, W), frames.dtype),
            jax.ShapeDtypeStruct((C, T, H, W), frames.dtype),
        ],
        compiler_params=pltpu.CompilerParams(vmem_limit_bytes=100 * 1024 * 1024),
    )(frames)
    return (slow, fast)
